# Initial kernel scaffold; baseline (speedup 1.0000x reference)
import jax
import jax.numpy as jnp
from jax.experimental import pallas as pl


def kernel(predicted, centroids, count, distances, class_weights_matrix, target, epoch):
    raise NotImplementedError("write your pallas kernel here")


# speedup vs baseline: 2.8669x; 2.8669x over previous
import functools

import jax
import jax.numpy as jnp
from jax import lax
from jax.experimental import pallas as pl
from jax.experimental.pallas import tpu as pltpu
from jax.experimental.pallas import tpu_sc as plsc

C = 10
N = 16384
D = 64
L = 16
NW = 32
RPW = N // NW
NCH = D // L
NG = RPW // L
CD = C * D


def _sqrt16(a):
    i = lax.bitcast_convert_type(a, jnp.int32)
    z = lax.bitcast_convert_type(jnp.int32(0x5F3759DF) - (i >> 1), jnp.float32)
    for _ in range(3):
        z = z * (1.5 - (0.5 * a) * z * z)
    return a * z


def _pass1_body(
    pred_hbm, tgt_hbm, cent_hbm, countp_hbm,
    partA_hbm, partB_hbm,
    pred_v, tgt_v, cent_v, countp_v, invc_v,
    acc_v, accvl_v, accvec_v,
):
    wid = lax.axis_index("s") * 2 + lax.axis_index("c")
    base = wid * RPW

    pltpu.sync_copy(pred_hbm.at[pl.ds(base * D, RPW * D)], pred_v)
    pltpu.sync_copy(tgt_hbm.at[pl.ds(base, RPW)], tgt_v)
    pltpu.sync_copy(cent_hbm, cent_v)
    pltpu.sync_copy(countp_hbm, countp_v)

    invc_v[...] = 1.0 / countp_v[...]

    zv = jnp.zeros((L,), jnp.float32)
    for q in range(CD // L):
        acc_v[pl.ds(q * L, L)] = zv
    for c in range(C):
        accvl_v[pl.ds(c * L, L)] = zv

    lane_iota = lax.iota(jnp.int32, L)
    invc_all = invc_v[...]

    def grp_body(g, carry):
        tvec = tgt_v[pl.ds(g * L, L)]
        invvec = jnp.zeros((L,), jnp.float32)
        for c in range(C):
            invvec = jnp.where(tvec == c, invc_all[c], invvec)
        svec = jnp.zeros((L,), jnp.float32)
        for lane in range(L):
            cls = tvec[lane]
            inv = invvec[lane]
            rb = (g * L + lane) * D
            cb = cls * D
            s2 = jnp.zeros((L,), jnp.float32)
            for k in range(NCH):
                x = pred_v[pl.ds(rb + k * L, L)]
                pr = x * inv
                diff = cent_v[pl.ds(cb + k * L, L)] - pr
                s2 = s2 + diff * diff
                plsc.addupdate(acc_v.at[pl.ds(cb + k * L, L)], pr)
            svec = jnp.where(lane_iota == lane, jnp.sum(s2), svec)
        nrm = _sqrt16(svec)
        for c in range(C):
            contrib = jnp.where(tvec == c, nrm, 0.0)
            plsc.addupdate(accvl_v.at[pl.ds(c * L, L)], contrib)
        return carry

    lax.fori_loop(0, NG, grp_body, 0)

    avec = jnp.zeros((L,), jnp.float32)
    for c in range(C):
        avec = jnp.where(lane_iota == c, jnp.sum(accvl_v[pl.ds(c * L, L)]), avec)
    accvec_v[...] = avec

    pltpu.sync_copy(acc_v, partA_hbm.at[pl.ds(wid * CD, CD)])
    pltpu.sync_copy(accvec_v, partB_hbm.at[pl.ds(wid * L, L)])


def _pass2_body(
    cent_hbm, countp_hbm, distp_hbm, wp_hbm, partA_hbm, partB_hbm,
    out_hbm,
    cent_v, countp_v, invc_v, distp_v, wp_v,
    allA_v, allB_v, acc_v, tot_v, out_v,
):
    wid = lax.axis_index("s") * 2 + lax.axis_index("c")

    @pl.when(wid == 0)
    def _():
        pltpu.sync_copy(cent_hbm, cent_v)
        pltpu.sync_copy(countp_hbm, countp_v)
        pltpu.sync_copy(distp_hbm, distp_v)
        pltpu.sync_copy(wp_hbm, wp_v)
        pltpu.sync_copy(partA_hbm, allA_v)
        pltpu.sync_copy(partB_hbm, allB_v)

        invc_v[...] = 1.0 / countp_v[...]
        lane_iota = lax.iota(jnp.int32, L)

        for q in range(CD // L):
            acc_v[pl.ds(q * L, L)] = cent_v[pl.ds(q * L, L)]

        def comb_body(w, carry):
            wb = w * CD
            for q in range(CD // L):
                plsc.addupdate(acc_v.at[pl.ds(q * L, L)],
                               allA_v[pl.ds(wb + q * L, L)])
            return carry

        lax.fori_loop(0, NW, comb_body, 0)

        sabs = jnp.zeros((L,), jnp.float32)
        for q in range(CD // L):
            sabs = sabs + jnp.abs(acc_v[pl.ds(q * L, L)])
        sabs_s = jnp.sum(sabs)

        def svec_body(w, carry):
            plsc.addupdate(distp_v.at[:], allB_v[pl.ds(w * L, L)])
            return carry

        lax.fori_loop(0, NW, svec_body, 0)
        s_vec = _sqrt16(distp_v[...]) * invc_v[...]

        tot = jnp.zeros((L,), jnp.float32)
        for i in range(C):
            ib = i * D
            ci = [acc_v[pl.ds(ib + k * L, L)] for k in range(NCH)]
            d2row = jnp.zeros((L,), jnp.float32)
            for j in range(C):
                if j == i:
                    continue
                s2 = jnp.zeros((L,), jnp.float32)
                for k in range(NCH):
                    dv = ci[k] - acc_v[pl.ds(j * D + k * L, L)]
                    s2 = s2 + dv * dv
                d2row = jnp.where(lane_iota == j, jnp.sum(s2), d2row)
            mask = (lane_iota < C) & (lane_iota != i)
            mrow = _sqrt16(jnp.where(mask, d2row, 1.0))
            numer = wp_v[pl.ds(i * L, L)] * (s_vec[i] + s_vec)
            term = jnp.where(mask, numer / mrow, 0.0)
            tot = tot + term
        tot_v[...] = tot

        total_s = jnp.sum(tot_v[...])
        total_vec = jnp.zeros((L,), jnp.float32) + total_s
        sabs_vec = jnp.zeros((L,), jnp.float32) + sabs_s
        loss_vec = total_vec / float(C) * float(C - 1) + sabs_vec / 1000000.0
        out_v[...] = loss_vec
        pltpu.sync_copy(out_v, out_hbm)


@jax.jit
def _db_loss(pred, tgt, cent, countp, distp, wp):
    mesh = plsc.VectorSubcoreMesh(core_axis_name="c", subcore_axis_name="s")
    params = pltpu.CompilerParams(needs_layout_passes=False)

    pass1 = functools.partial(
        pl.kernel,
        out_type=[
            jax.ShapeDtypeStruct((NW * CD,), jnp.float32),
            jax.ShapeDtypeStruct((NW * L,), jnp.float32),
        ],
        mesh=mesh,
        compiler_params=params,
        scratch_types=[
            pltpu.VMEM((RPW * D,), jnp.float32),
            pltpu.VMEM((RPW,), jnp.int32),
            pltpu.VMEM((CD,), jnp.float32),
            pltpu.VMEM((L,), jnp.float32),
            pltpu.VMEM((L,), jnp.float32),
            pltpu.VMEM((CD,), jnp.float32),
            pltpu.VMEM((C * L,), jnp.float32),
            pltpu.VMEM((L,), jnp.float32),
        ],
    )(_pass1_body)
    partA, partB = pass1(pred, tgt, cent, countp)

    pass2 = functools.partial(
        pl.kernel,
        out_type=jax.ShapeDtypeStruct((L,), jnp.float32),
        mesh=mesh,
        compiler_params=params,
        scratch_types=[
            pltpu.VMEM((CD,), jnp.float32),
            pltpu.VMEM((L,), jnp.float32),
            pltpu.VMEM((L,), jnp.float32),
            pltpu.VMEM((L,), jnp.float32),
            pltpu.VMEM((C * L,), jnp.float32),
            pltpu.VMEM((NW * CD,), jnp.float32),
            pltpu.VMEM((NW * L,), jnp.float32),
            pltpu.VMEM((CD,), jnp.float32),
            pltpu.VMEM((L,), jnp.float32),
            pltpu.VMEM((L,), jnp.float32),
        ],
    )(_pass2_body)
    return pass2(cent, countp, distp, wp, partA, partB)


def kernel(predicted, centroids, count, distances, class_weights_matrix, target, epoch):
    countp = jnp.concatenate([count[:, 0], jnp.ones((L - C,), jnp.float32)])
    distp = jnp.concatenate([distances[:, 0], jnp.zeros((L - C,), jnp.float32)])
    wp = jnp.pad(class_weights_matrix, ((0, 0), (0, L - C))).reshape(C * L)
    out = _db_loss(predicted.reshape(N * D), target.astype(jnp.int32),
                   centroids.reshape(CD), countp, distp, wp)
    return out[:1]
